# baseline (speedup 1.0000x reference)
import dataclasses
import functools

import jax
import jax.numpy as jnp
from jax import lax
from jax.experimental import pallas as pl
from jax.experimental.pallas import tpu as pltpu
from jax.experimental.pallas import tpu_sc as plsc

NN = 10000
NE = 320000
D = 128
NL = 4

NC = 2
NS = 16
NW = NC * NS
EPT = NE // NW
CW = 16
NCHUNK = EPT // CW
NPAIR = (NCHUNK - 1) // 2
NNP = 10240
ARPT = NNP // NS


def _node_mm_body(h_ref, w_ref, b_ref, p0_ref, p1_ref, u_ref):
    z = jnp.dot(h_ref[...], w_ref[...], preferred_element_type=jnp.float32,
                precision=lax.Precision.HIGHEST) + b_ref[...]
    p0_ref[...] = z[:, :4 * D].astype(jnp.bfloat16)
    p1_ref[...] = z[:, :2 * D].astype(jnp.bfloat16)
    u_ref[...] = z[:, 4 * D:]


def _node_mm(h, w, b):
    bm = 1000
    return pl.pallas_call(
        _node_mm_body,
        grid=(NN // bm,),
        in_specs=[pl.BlockSpec((bm, D), lambda i: (i, 0)),
                  pl.BlockSpec((D, 5 * D), lambda i: (0, 0)),
                  pl.BlockSpec((1, 5 * D), lambda i: (0, 0))],
        out_specs=[pl.BlockSpec((bm, 4 * D), lambda i: (i, 0)),
                   pl.BlockSpec((bm, 2 * D), lambda i: (i, 0)),
                   pl.BlockSpec((bm, D), lambda i: (i, 0))],
        out_shape=[jax.ShapeDtypeStruct((NN, 4 * D), jnp.bfloat16),
                   jax.ShapeDtypeStruct((NN, 2 * D), jnp.bfloat16),
                   jax.ShapeDtypeStruct((NN, D), jnp.float32)],
    )(h, w, b)


def _edge_mm_body(e_ref, w_ref, b_ref, ce_ref):
    ce_ref[:, :D] = jnp.dot(e_ref[...], w_ref[...], preferred_element_type=jnp.float32,
                            precision=lax.Precision.HIGHEST) + b_ref[...]
    ce_ref[:, D:] = e_ref[...]


def _edge_mm(e, w, b):
    bm = 4000
    return pl.pallas_call(
        _edge_mm_body,
        grid=(NE // bm,),
        in_specs=[pl.BlockSpec((bm, D), lambda i: (i, 0)),
                  pl.BlockSpec((D, D), lambda i: (0, 0)),
                  pl.BlockSpec((1, D), lambda i: (0, 0))],
        out_specs=pl.BlockSpec((bm, 2 * D), lambda i: (i, 0)),
        out_shape=jax.ShapeDtypeStruct((NE, 2 * D), jnp.float32),
    )(e, w, b)


def _finish_body(acc_ref, u_ref, hin_ref, o_ref):
    o_ref[...] = (jnp.maximum(acc_ref[0] + acc_ref[1] + u_ref[...], 0.0)
                  + hin_ref[...])


def _finish(acc, u, h):
    bm = 2000
    return pl.pallas_call(
        _finish_body,
        grid=(NN // bm,),
        in_specs=[pl.BlockSpec((NC, bm, D), lambda i: (0, i, 0)),
                  pl.BlockSpec((bm, D), lambda i: (i, 0)),
                  pl.BlockSpec((bm, D), lambda i: (i, 0))],
        out_specs=pl.BlockSpec((bm, D), lambda i: (i, 0)),
        out_shape=jax.ShapeDtypeStruct((NN, D), jnp.float32),
    )(acc, u, h)


def _edge_pass_body(eix_hbm, zs_hbm, p0_hbm, p1_hbm, ce_hbm,
                    ef_hbm, acc_hbm,
                    i0a, g0a, g1a, cea, efa, msa,
                    i0b, g0b, g1b, ceb, efb, msb,
                    accsh,
                    si0a, sg0a, sg1a, scea, sefa, smsa,
                    si0b, sg0b, sg1b, sceb, sefb, smsb):
    cid = lax.axis_index("c")
    sid = lax.axis_index("s")
    wid = cid * NS + sid
    ebase = wid * EPT

    seta = (i0a, None, g0a, g1a, cea, efa, msa,
            si0a, None, sg0a, sg1a, scea, sefa, smsa)
    setb = (i0b, None, g0b, g1b, ceb, efb, msb,
            si0b, None, sg0b, sg1b, sceb, sefb, smsb)

    def start_idx(c, st):
        cg = wid * NCHUNK + c
        pltpu.async_copy(eix_hbm.at[pl.ds(cg * 2 * CW, 2 * CW)], st[0], st[7])

    def wait_idx(st):
        pltpu.make_async_copy(eix_hbm.at[pl.ds(0, 2 * CW)], st[0], st[7]).wait()

    def start_gather(c, st):
        eb = ebase + c * CW
        pltpu.async_copy(p0_hbm.at[st[0].at[pl.ds(0, CW)]], st[2], st[9])
        pltpu.async_copy(p1_hbm.at[st[0].at[pl.ds(CW, CW)]], st[3], st[10])
        pltpu.async_copy(ce_hbm.at[pl.ds(eb, CW)], st[4], st[11])

    def wait_gather(st):
        pltpu.make_async_copy(p0_hbm.at[st[0].at[pl.ds(0, CW)]], st[2], st[9]).wait()
        pltpu.make_async_copy(p1_hbm.at[st[0].at[pl.ds(CW, CW)]], st[3], st[10]).wait()
        pltpu.make_async_copy(ce_hbm.at[pl.ds(0, CW)], st[4], st[11]).wait()

    def wait_outs(st):
        pltpu.make_async_copy(st[5], ef_hbm.at[pl.ds(0, CW)], st[12]).wait()
        pltpu.make_async_copy(st[6], accsh.at[st[0].at[pl.ds(CW, CW)]], st[13]).wait()

    def compute(c, st, iv):
        g0, g1, ce, ef, ms = st[2], st[3], st[4], st[5], st[6]

        def unpk(w):
            return plsc.unpack(plsc.bitcast(w, jnp.bfloat16),
                               format=plsc.PackFormat.INTERLEAVED)

        @plsc.parallel_loop(0, CW, unroll=2)
        def _(i):
            W32 = D // 2
            for g in range(D // 32):
                a0 = unpk(g0[i, pl.ds(g * 16, 16)])
                b0 = unpk(g0[i, pl.ds(W32 + g * 16, 16)])
                vf = unpk(g0[i, pl.ds(2 * W32 + g * 16, 16)])
                vb = unpk(g0[i, pl.ds(3 * W32 + g * 16, 16)])
                a1 = unpk(g1[i, pl.ds(g * 16, 16)])
                b1 = unpk(g1[i, pl.ds(W32 + g * 16, 16)])
                for hh in range(2):
                    sl = pl.ds(g * 32 + hh * 16, 16)
                    cc = ce[i, pl.ds(g * 32 + hh * 16, 16)]
                    ee = ce[i, pl.ds(D + g * 32 + hh * 16, 16)]
                    efv = jnp.maximum(a0[hh] + b1[hh] + cc, 0.0) + ee
                    ebv = jnp.maximum(a1[hh] + b0[hh] + cc, 0.0) + ee
                    gf = 1.0 / ((1.0 + 1e-6) + 1e-6 * jnp.exp(-efv))
                    gb = 1.0 / ((1.0 + 1e-6) + 1e-6 * jnp.exp(-ebv))
                    ef[i, sl] = efv
                    ms[i, sl] = vf[hh] * gf + vb[hh] * gb

        eb = ebase + c * CW
        pltpu.async_copy(ef, ef_hbm.at[pl.ds(eb, CW)], st[12])
        pltpu.async_copy(ms, accsh.at[plsc.Indices(iv)], st[13], add=True)

    pltpu.sync_copy(zs_hbm.at[pl.ds(sid * ARPT, ARPT)],
                    accsh.at[pl.ds(sid * ARPT, ARPT)])

    plsc.subcore_barrier()

    start_idx(0, seta)
    start_idx(1, setb)
    wait_idx(seta)
    start_gather(0, seta)

    @pl.loop(0, NPAIR)
    def _(t):
        c = 2 * t
        wait_gather(seta)
        iva = i0a[pl.ds(CW, CW)]

        @pl.when(t >= 1)
        def _():
            wait_outs(seta)

        start_idx(c + 2, seta)
        wait_idx(setb)
        start_gather(c + 1, setb)
        compute(c, seta, iva)

        wait_gather(setb)
        ivb = i0b[pl.ds(CW, CW)]

        @pl.when(t >= 1)
        def _():
            wait_outs(setb)

        @pl.when(t < NPAIR - 1)
        def _():
            start_idx(c + 3, setb)

        wait_idx(seta)
        start_gather(c + 2, seta)
        compute(c + 1, setb, ivb)

    wait_gather(seta)
    iva = i0a[pl.ds(CW, CW)]
    wait_outs(seta)
    compute(NCHUNK - 1, seta, iva)
    wait_outs(setb)
    wait_outs(seta)

    plsc.subcore_barrier()

    r0 = sid * ARPT
    pltpu.sync_copy(accsh.at[pl.ds(r0, ARPT)],
                    acc_hbm.at[cid].at[pl.ds(r0, ARPT)])


@jax.jit
def _edge_pass(eix, zs, p0, p1, ce):
    mesh = plsc.VectorSubcoreMesh(core_axis_name="c", subcore_axis_name="s",
                                  num_cores=NC, num_subcores=NS)
    buf_set = [
        pltpu.VMEM((2 * CW,), jnp.int32),
        pltpu.VMEM((CW, 2 * D), jnp.float32),
        pltpu.VMEM((CW, D), jnp.float32),
        pltpu.VMEM((CW, 2 * D), jnp.float32),
        pltpu.VMEM((CW, D), jnp.float32),
        pltpu.VMEM((CW, D), jnp.float32),
    ]
    cp = pltpu.CompilerParams()
    if "needs_layout_passes" in pltpu.CompilerParams.__dataclass_fields__:
        cp = dataclasses.replace(cp, needs_layout_passes=False)
    f = pl.kernel(
        _edge_pass_body,
        out_type=[jax.ShapeDtypeStruct((NE, D), jnp.float32),
                  jax.ShapeDtypeStruct((NC, NNP, D), jnp.float32)],
        mesh=mesh,
        compiler_params=cp,
        scratch_types=(buf_set + buf_set
                       + [pltpu.VMEM_SHARED((NNP, D), jnp.float32)]
                       + [pltpu.SemaphoreType.DMA] * 12),
    )
    return f(eix, zs, p0, p1, ce)


def kernel(edge_index, h, e, WA, bA, WB, bB, WC, bC, WU, bU, WVf, bVf,
           WVb, bVb):
    ei0 = edge_index[0]
    ei1 = edge_index[1]
    def ileave(w):
        n = w.shape[-1]
        p = jnp.arange(n).reshape(n // 32, 2, 16).transpose(0, 2, 1).reshape(-1)
        return w[..., p]

    wcat = jnp.concatenate([
        ileave(WA.transpose(0, 2, 1)), ileave(WB.transpose(0, 2, 1)),
        ileave(WVf.transpose(0, 2, 1)), ileave(WVb.transpose(0, 2, 1)),
        WU.transpose(0, 2, 1)], axis=2)
    bcat = jnp.concatenate([ileave(bA), ileave(bB), ileave(bVf), ileave(bVb),
                            bU], axis=1)
    wct = WC.transpose(0, 2, 1)

    eix = jnp.concatenate([ei0.reshape(-1, CW), ei1.reshape(-1, CW)],
                          axis=1).reshape(-1)
    zs = jnp.zeros((NNP, D), jnp.float32)

    for l in range(NL):
        p0, p1, u = _node_mm(h, wcat[l], bcat[l][None])
        ce = _edge_mm(e, wct[l], bC[l][None])
        p0f = lax.bitcast_convert_type(p0.reshape(NN, 2 * D, 2), jnp.float32)
        p1f = lax.bitcast_convert_type(p1.reshape(NN, D, 2), jnp.float32)
        ef, acc = _edge_pass(eix, zs, p0f, p1f, ce)
        h = _finish(acc, u, h)
        e = ef
    return (h, e)


# speedup vs baseline: 1.2970x; 1.2970x over previous
import functools

import jax
import jax.numpy as jnp
from jax import lax
from jax.experimental import pallas as pl
from jax.experimental.pallas import tpu as pltpu
from jax.experimental.pallas import tpu_sc as plsc

NN = 10000
NE = 320000
D = 128
NL = 4

NC = 2
NS = 16
NW = NC * NS
EPT = NE // NW
CW = 16
NCHUNK = EPT // CW
NPAIR = (NCHUNK - 1) // 2
NNP = 10240
ARPT = NNP // NS


def _node_mm_body(h_ref, w_ref, b_ref, p0_ref, p1_ref, u_ref):
    z = jnp.dot(h_ref[...], w_ref[...], preferred_element_type=jnp.float32,
                precision=lax.Precision.HIGHEST) + b_ref[...]
    p0_ref[...] = z[:, :4 * D]
    p1_ref[...] = z[:, :2 * D]
    u_ref[...] = z[:, 4 * D:]


def _node_mm(h, w, b):
    bm = 1000
    return pl.pallas_call(
        _node_mm_body,
        grid=(NN // bm,),
        in_specs=[pl.BlockSpec((bm, D), lambda i: (i, 0)),
                  pl.BlockSpec((D, 5 * D), lambda i: (0, 0)),
                  pl.BlockSpec((1, 5 * D), lambda i: (0, 0))],
        out_specs=[pl.BlockSpec((bm, 4 * D), lambda i: (i, 0)),
                   pl.BlockSpec((bm, 2 * D), lambda i: (i, 0)),
                   pl.BlockSpec((bm, D), lambda i: (i, 0))],
        out_shape=[jax.ShapeDtypeStruct((NN, 4 * D), jnp.float32),
                   jax.ShapeDtypeStruct((NN, 2 * D), jnp.float32),
                   jax.ShapeDtypeStruct((NN, D), jnp.float32)],
    )(h, w, b)


def _edge_mm_body(e_ref, w_ref, b_ref, ce_ref):
    ce_ref[:, :D] = jnp.dot(e_ref[...], w_ref[...], preferred_element_type=jnp.float32,
                            precision=lax.Precision.HIGHEST) + b_ref[...]
    ce_ref[:, D:] = e_ref[...]


def _edge_mm(e, w, b):
    bm = 4000
    return pl.pallas_call(
        _edge_mm_body,
        grid=(NE // bm,),
        in_specs=[pl.BlockSpec((bm, D), lambda i: (i, 0)),
                  pl.BlockSpec((D, D), lambda i: (0, 0)),
                  pl.BlockSpec((1, D), lambda i: (0, 0))],
        out_specs=pl.BlockSpec((bm, 2 * D), lambda i: (i, 0)),
        out_shape=jax.ShapeDtypeStruct((NE, 2 * D), jnp.float32),
    )(e, w, b)


def _finish_body(acc_ref, u_ref, hin_ref, o_ref):
    o_ref[...] = (jnp.maximum(acc_ref[0] + acc_ref[1] + u_ref[...], 0.0)
                  + hin_ref[...])


def _finish(acc, u, h):
    bm = 2000
    return pl.pallas_call(
        _finish_body,
        grid=(NN // bm,),
        in_specs=[pl.BlockSpec((NC, bm, D), lambda i: (0, i, 0)),
                  pl.BlockSpec((bm, D), lambda i: (i, 0)),
                  pl.BlockSpec((bm, D), lambda i: (i, 0))],
        out_specs=pl.BlockSpec((bm, D), lambda i: (i, 0)),
        out_shape=jax.ShapeDtypeStruct((NN, D), jnp.float32),
    )(acc, u, h)


def _edge_pass_body(eix_hbm, zs_hbm, p0_hbm, p1_hbm, ce_hbm,
                    ef_hbm, acc_hbm,
                    i0a, g0a, g1a, cea, efa, msa,
                    i0b, g0b, g1b, ceb, efb, msb,
                    accsh,
                    si0a, sg0a, sg1a, scea, sefa, smsa,
                    si0b, sg0b, sg1b, sceb, sefb, smsb):
    cid = lax.axis_index("c")
    sid = lax.axis_index("s")
    wid = cid * NS + sid
    ebase = wid * EPT

    seta = (i0a, None, g0a, g1a, cea, efa, msa,
            si0a, None, sg0a, sg1a, scea, sefa, smsa)
    setb = (i0b, None, g0b, g1b, ceb, efb, msb,
            si0b, None, sg0b, sg1b, sceb, sefb, smsb)

    def start_idx(c, st):
        cg = wid * NCHUNK + c
        pltpu.async_copy(eix_hbm.at[pl.ds(cg * 2 * CW, 2 * CW)], st[0], st[7])

    def wait_idx(st):
        pltpu.make_async_copy(eix_hbm.at[pl.ds(0, 2 * CW)], st[0], st[7]).wait()

    def start_gather(c, st):
        eb = ebase + c * CW
        pltpu.async_copy(p0_hbm.at[st[0].at[pl.ds(0, CW)]], st[2], st[9])
        pltpu.async_copy(p1_hbm.at[st[0].at[pl.ds(CW, CW)]], st[3], st[10])
        pltpu.async_copy(ce_hbm.at[pl.ds(eb, CW)], st[4], st[11])

    def wait_gather(st):
        pltpu.make_async_copy(p0_hbm.at[st[0].at[pl.ds(0, CW)]], st[2], st[9]).wait()
        pltpu.make_async_copy(p1_hbm.at[st[0].at[pl.ds(CW, CW)]], st[3], st[10]).wait()
        pltpu.make_async_copy(ce_hbm.at[pl.ds(0, CW)], st[4], st[11]).wait()

    def wait_outs(st):
        pltpu.make_async_copy(st[5], ef_hbm.at[pl.ds(0, CW)], st[12]).wait()
        pltpu.make_async_copy(st[6], accsh.at[st[0].at[pl.ds(CW, CW)]], st[13]).wait()

    def compute(c, st, iv):
        g0, g1, ce, ef, ms = st[2], st[3], st[4], st[5], st[6]

        @plsc.parallel_loop(0, CW, unroll=2)
        def _(i):
            for g in range(D // 16):
                sl = pl.ds(g * 16, 16)
                a0 = g0[i, sl]
                b0 = g0[i, pl.ds(D + g * 16, 16)]
                vf = g0[i, pl.ds(2 * D + g * 16, 16)]
                vb = g0[i, pl.ds(3 * D + g * 16, 16)]
                a1 = g1[i, sl]
                b1 = g1[i, pl.ds(D + g * 16, 16)]
                cc = ce[i, sl]
                ee = ce[i, pl.ds(D + g * 16, 16)]
                efv = jnp.maximum(a0 + b1 + cc, 0.0) + ee
                ebv = jnp.maximum(a1 + b0 + cc, 0.0) + ee
                tf = jnp.minimum(jnp.exp(-efv), 1e18)
                tb = jnp.minimum(jnp.exp(-ebv), 1e18)
                df = (1.0 + 1e-6) + 1e-6 * tf
                db = (1.0 + 1e-6) + 1e-6 * tb
                ef[i, sl] = efv
                ms[i, sl] = (vf * db + vb * df) / (df * db)

        eb = ebase + c * CW
        pltpu.async_copy(ef, ef_hbm.at[pl.ds(eb, CW)], st[12])
        pltpu.async_copy(ms, accsh.at[plsc.Indices(iv)], st[13], add=True)

    pltpu.sync_copy(zs_hbm.at[pl.ds(sid * ARPT, ARPT)],
                    accsh.at[pl.ds(sid * ARPT, ARPT)])

    plsc.subcore_barrier()

    start_idx(0, seta)
    start_idx(1, setb)
    wait_idx(seta)
    start_gather(0, seta)

    @pl.loop(0, NPAIR)
    def _(t):
        c = 2 * t
        wait_gather(seta)
        iva = i0a[pl.ds(CW, CW)]

        @pl.when(t >= 1)
        def _():
            wait_outs(seta)

        start_idx(c + 2, seta)
        wait_idx(setb)
        start_gather(c + 1, setb)
        compute(c, seta, iva)

        wait_gather(setb)
        ivb = i0b[pl.ds(CW, CW)]

        @pl.when(t >= 1)
        def _():
            wait_outs(setb)

        @pl.when(t < NPAIR - 1)
        def _():
            start_idx(c + 3, setb)

        wait_idx(seta)
        start_gather(c + 2, seta)
        compute(c + 1, setb, ivb)

    wait_gather(seta)
    iva = i0a[pl.ds(CW, CW)]
    wait_outs(seta)
    compute(NCHUNK - 1, seta, iva)
    wait_outs(setb)
    wait_outs(seta)

    plsc.subcore_barrier()

    r0 = sid * ARPT
    pltpu.sync_copy(accsh.at[pl.ds(r0, ARPT)],
                    acc_hbm.at[cid].at[pl.ds(r0, ARPT)])


@jax.jit
def _edge_pass(eix, zs, p0, p1, ce):
    mesh = plsc.VectorSubcoreMesh(core_axis_name="c", subcore_axis_name="s",
                                  num_cores=NC, num_subcores=NS)
    buf_set = [
        pltpu.VMEM((2 * CW,), jnp.int32),
        pltpu.VMEM((CW, 4 * D), jnp.float32),
        pltpu.VMEM((CW, 2 * D), jnp.float32),
        pltpu.VMEM((CW, 2 * D), jnp.float32),
        pltpu.VMEM((CW, D), jnp.float32),
        pltpu.VMEM((CW, D), jnp.float32),
    ]
    f = pl.kernel(
        _edge_pass_body,
        out_type=[jax.ShapeDtypeStruct((NE, D), jnp.float32),
                  jax.ShapeDtypeStruct((NC, NNP, D), jnp.float32)],
        mesh=mesh,
        scratch_types=(buf_set + buf_set
                       + [pltpu.VMEM_SHARED((NNP, D), jnp.float32)]
                       + [pltpu.SemaphoreType.DMA] * 12),
    )
    return f(eix, zs, p0, p1, ce)


def kernel(edge_index, h, e, WA, bA, WB, bB, WC, bC, WU, bU, WVf, bVf,
           WVb, bVb):
    ei0 = edge_index[0]
    ei1 = edge_index[1]
    wcat = jnp.concatenate([
        WA.transpose(0, 2, 1), WB.transpose(0, 2, 1),
        WVf.transpose(0, 2, 1), WVb.transpose(0, 2, 1),
        WU.transpose(0, 2, 1)], axis=2)
    bcat = jnp.concatenate([bA, bB, bVf, bVb, bU], axis=1)
    wct = WC.transpose(0, 2, 1)

    eix = jnp.concatenate([ei0.reshape(-1, CW), ei1.reshape(-1, CW)],
                          axis=1).reshape(-1)
    zs = jnp.zeros((NNP, D), jnp.float32)

    for l in range(NL):
        p0, p1, u = _node_mm(h, wcat[l], bcat[l][None])
        ce = _edge_mm(e, wct[l], bC[l][None])
        ef, acc = _edge_pass(eix, zs, p0, p1, ce)
        h = _finish(acc, u, h)
        e = ef
    return (h, e)


# speedup vs baseline: 1.3522x; 1.0426x over previous
import functools

import jax
import jax.numpy as jnp
from jax import lax
from jax.experimental import pallas as pl
from jax.experimental.pallas import tpu as pltpu
from jax.experimental.pallas import tpu_sc as plsc

NN = 10000
NE = 320000
D = 128
NL = 4

NC = 2
NS = 16
NW = NC * NS
EPT = NE // NW
CW = 16
NCHUNK = EPT // CW
NPAIR = (NCHUNK - 1) // 2
NNP = 10240
ARPT = NNP // NS


def _node_mm_body(h_ref, w_ref, b_ref, p0_ref, p1_ref, u_ref):
    z = jnp.dot(h_ref[...], w_ref[...], preferred_element_type=jnp.float32,
                precision=lax.Precision.HIGHEST) + b_ref[...]
    p0_ref[...] = z[:, :4 * D]
    p1_ref[...] = z[:, :2 * D]
    u_ref[...] = z[:, 4 * D:]


def _node_mm(h, w, b):
    bm = 1000
    return pl.pallas_call(
        _node_mm_body,
        grid=(NN // bm,),
        in_specs=[pl.BlockSpec((bm, D), lambda i: (i, 0)),
                  pl.BlockSpec((D, 5 * D), lambda i: (0, 0)),
                  pl.BlockSpec((1, 5 * D), lambda i: (0, 0))],
        out_specs=[pl.BlockSpec((bm, 4 * D), lambda i: (i, 0)),
                   pl.BlockSpec((bm, 2 * D), lambda i: (i, 0)),
                   pl.BlockSpec((bm, D), lambda i: (i, 0))],
        out_shape=[jax.ShapeDtypeStruct((NN, 4 * D), jnp.float32),
                   jax.ShapeDtypeStruct((NN, 2 * D), jnp.float32),
                   jax.ShapeDtypeStruct((NN, D), jnp.float32)],
    )(h, w, b)


def _edge_mm_body(e_ref, w_ref, b_ref, ce_ref):
    ce_ref[:, :D] = jnp.dot(e_ref[...], w_ref[...], preferred_element_type=jnp.float32,
                            precision=lax.Precision.HIGHEST) + b_ref[...]
    ce_ref[:, D:] = e_ref[...]


def _edge_mm(e, w, b):
    bm = 4000
    return pl.pallas_call(
        _edge_mm_body,
        grid=(NE // bm,),
        in_specs=[pl.BlockSpec((bm, D), lambda i: (i, 0)),
                  pl.BlockSpec((D, D), lambda i: (0, 0)),
                  pl.BlockSpec((1, D), lambda i: (0, 0))],
        out_specs=pl.BlockSpec((bm, 2 * D), lambda i: (i, 0)),
        out_shape=jax.ShapeDtypeStruct((NE, 2 * D), jnp.float32),
    )(e, w, b)


def _node_mm_fused_body(acc_ref, u_ref, hp_ref, w_ref, b_ref,
                        p0_ref, p1_ref, u_ref_o, h_ref_o):
    hnew = (jnp.maximum(acc_ref[0] + acc_ref[1] + u_ref[...], 0.0)
            + hp_ref[...])
    z = jnp.dot(hnew, w_ref[...], preferred_element_type=jnp.float32,
                precision=lax.Precision.HIGHEST) + b_ref[...]
    p0_ref[...] = z[:, :4 * D]
    p1_ref[...] = z[:, :2 * D]
    u_ref_o[...] = z[:, 4 * D:]
    h_ref_o[...] = hnew


def _node_mm_fused(acc, u, hp, w, b):
    bm = 1000
    return pl.pallas_call(
        _node_mm_fused_body,
        grid=(NN // bm,),
        in_specs=[pl.BlockSpec((NC, bm, D), lambda i: (0, i, 0)),
                  pl.BlockSpec((bm, D), lambda i: (i, 0)),
                  pl.BlockSpec((bm, D), lambda i: (i, 0)),
                  pl.BlockSpec((D, 5 * D), lambda i: (0, 0)),
                  pl.BlockSpec((1, 5 * D), lambda i: (0, 0))],
        out_specs=[pl.BlockSpec((bm, 4 * D), lambda i: (i, 0)),
                   pl.BlockSpec((bm, 2 * D), lambda i: (i, 0)),
                   pl.BlockSpec((bm, D), lambda i: (i, 0)),
                   pl.BlockSpec((bm, D), lambda i: (i, 0))],
        out_shape=[jax.ShapeDtypeStruct((NN, 4 * D), jnp.float32),
                   jax.ShapeDtypeStruct((NN, 2 * D), jnp.float32),
                   jax.ShapeDtypeStruct((NN, D), jnp.float32),
                   jax.ShapeDtypeStruct((NN, D), jnp.float32)],
    )(acc, u, hp, w, b)


def _finish_body(acc_ref, u_ref, hin_ref, o_ref):
    o_ref[...] = (jnp.maximum(acc_ref[0] + acc_ref[1] + u_ref[...], 0.0)
                  + hin_ref[...])


def _finish(acc, u, h):
    bm = 2000
    return pl.pallas_call(
        _finish_body,
        grid=(NN // bm,),
        in_specs=[pl.BlockSpec((NC, bm, D), lambda i: (0, i, 0)),
                  pl.BlockSpec((bm, D), lambda i: (i, 0)),
                  pl.BlockSpec((bm, D), lambda i: (i, 0))],
        out_specs=pl.BlockSpec((bm, D), lambda i: (i, 0)),
        out_shape=jax.ShapeDtypeStruct((NN, D), jnp.float32),
    )(acc, u, h)


def _edge_pass_body(eix_hbm, zs_hbm, p0_hbm, p1_hbm, ce_hbm,
                    ef_hbm, acc_hbm,
                    i0a, g0a, g1a, cea, efa, msa,
                    i0b, g0b, g1b, ceb, efb, msb,
                    accsh,
                    si0a, sg0a, sg1a, scea, sefa, smsa,
                    si0b, sg0b, sg1b, sceb, sefb, smsb):
    cid = lax.axis_index("c")
    sid = lax.axis_index("s")
    wid = cid * NS + sid
    ebase = wid * EPT

    seta = (i0a, None, g0a, g1a, cea, efa, msa,
            si0a, None, sg0a, sg1a, scea, sefa, smsa)
    setb = (i0b, None, g0b, g1b, ceb, efb, msb,
            si0b, None, sg0b, sg1b, sceb, sefb, smsb)

    def start_idx(c, st):
        cg = wid * NCHUNK + c
        pltpu.async_copy(eix_hbm.at[pl.ds(cg * 2 * CW, 2 * CW)], st[0], st[7])

    def wait_idx(st):
        pltpu.make_async_copy(eix_hbm.at[pl.ds(0, 2 * CW)], st[0], st[7]).wait()

    def start_gather(c, st):
        eb = ebase + c * CW
        pltpu.async_copy(p0_hbm.at[st[0].at[pl.ds(0, CW)]], st[2], st[9])
        pltpu.async_copy(p1_hbm.at[st[0].at[pl.ds(CW, CW)]], st[3], st[10])
        pltpu.async_copy(ce_hbm.at[pl.ds(eb, CW)], st[4], st[11])

    def wait_gather(st):
        pltpu.make_async_copy(p0_hbm.at[st[0].at[pl.ds(0, CW)]], st[2], st[9]).wait()
        pltpu.make_async_copy(p1_hbm.at[st[0].at[pl.ds(CW, CW)]], st[3], st[10]).wait()
        pltpu.make_async_copy(ce_hbm.at[pl.ds(0, CW)], st[4], st[11]).wait()

    def wait_outs(st):
        pltpu.make_async_copy(st[5], ef_hbm.at[pl.ds(0, CW)], st[12]).wait()
        pltpu.make_async_copy(st[6], accsh.at[st[0].at[pl.ds(CW, CW)]], st[13]).wait()

    def compute(c, st, iv):
        g0, g1, ce, ef, ms = st[2], st[3], st[4], st[5], st[6]

        @plsc.parallel_loop(0, CW, unroll=2)
        def _(i):
            for g in range(D // 16):
                sl = pl.ds(g * 16, 16)
                a0 = g0[i, sl]
                b0 = g0[i, pl.ds(D + g * 16, 16)]
                vf = g0[i, pl.ds(2 * D + g * 16, 16)]
                vb = g0[i, pl.ds(3 * D + g * 16, 16)]
                a1 = g1[i, sl]
                b1 = g1[i, pl.ds(D + g * 16, 16)]
                cc = ce[i, sl]
                ee = ce[i, pl.ds(D + g * 16, 16)]
                efv = jnp.maximum(a0 + b1 + cc, 0.0) + ee
                ebv = jnp.maximum(a1 + b0 + cc, 0.0) + ee
                gf = 1.0 / ((1.0 + 1e-6) + 1e-6 * jnp.exp(-efv))
                gb = 1.0 / ((1.0 + 1e-6) + 1e-6 * jnp.exp(-ebv))
                ef[i, sl] = efv
                ms[i, sl] = vf * gf + vb * gb

        eb = ebase + c * CW
        pltpu.async_copy(ef, ef_hbm.at[pl.ds(eb, CW)], st[12])
        pltpu.async_copy(ms, accsh.at[plsc.Indices(iv)], st[13], add=True)

    pltpu.sync_copy(zs_hbm.at[pl.ds(sid * ARPT, ARPT)],
                    accsh.at[pl.ds(sid * ARPT, ARPT)])

    plsc.subcore_barrier()

    start_idx(0, seta)
    start_idx(1, setb)
    wait_idx(seta)
    start_gather(0, seta)

    @pl.loop(0, NPAIR)
    def _(t):
        c = 2 * t
        wait_gather(seta)
        iva = i0a[pl.ds(CW, CW)]

        @pl.when(t >= 1)
        def _():
            wait_outs(seta)

        start_idx(c + 2, seta)
        wait_idx(setb)
        start_gather(c + 1, setb)
        compute(c, seta, iva)

        wait_gather(setb)
        ivb = i0b[pl.ds(CW, CW)]

        @pl.when(t >= 1)
        def _():
            wait_outs(setb)

        @pl.when(t < NPAIR - 1)
        def _():
            start_idx(c + 3, setb)

        wait_idx(seta)
        start_gather(c + 2, seta)
        compute(c + 1, setb, ivb)

    wait_gather(seta)
    iva = i0a[pl.ds(CW, CW)]
    wait_outs(seta)
    compute(NCHUNK - 1, seta, iva)
    wait_outs(setb)
    wait_outs(seta)

    plsc.subcore_barrier()

    r0 = sid * ARPT
    pltpu.sync_copy(accsh.at[pl.ds(r0, ARPT)],
                    acc_hbm.at[cid].at[pl.ds(r0, ARPT)])


@jax.jit
def _edge_pass(eix, zs, p0, p1, ce):
    mesh = plsc.VectorSubcoreMesh(core_axis_name="c", subcore_axis_name="s",
                                  num_cores=NC, num_subcores=NS)
    buf_set = [
        pltpu.VMEM((2 * CW,), jnp.int32),
        pltpu.VMEM((CW, 4 * D), jnp.float32),
        pltpu.VMEM((CW, 2 * D), jnp.float32),
        pltpu.VMEM((CW, 2 * D), jnp.float32),
        pltpu.VMEM((CW, D), jnp.float32),
        pltpu.VMEM((CW, D), jnp.float32),
    ]
    f = pl.kernel(
        _edge_pass_body,
        out_type=[jax.ShapeDtypeStruct((NE, D), jnp.float32),
                  jax.ShapeDtypeStruct((NC, NNP, D), jnp.float32)],
        mesh=mesh,
        scratch_types=(buf_set + buf_set
                       + [pltpu.VMEM_SHARED((NNP, D), jnp.float32)]
                       + [pltpu.SemaphoreType.DMA] * 12),
    )
    return f(eix, zs, p0, p1, ce)


def kernel(edge_index, h, e, WA, bA, WB, bB, WC, bC, WU, bU, WVf, bVf,
           WVb, bVb):
    ei0 = edge_index[0]
    ei1 = edge_index[1]
    wcat = jnp.concatenate([
        WA.transpose(0, 2, 1), WB.transpose(0, 2, 1),
        WVf.transpose(0, 2, 1), WVb.transpose(0, 2, 1),
        WU.transpose(0, 2, 1)], axis=2)
    bcat = jnp.concatenate([bA, bB, bVf, bVb, bU], axis=1)
    wct = WC.transpose(0, 2, 1)

    eix = jnp.concatenate([ei0.reshape(-1, CW), ei1.reshape(-1, CW)],
                          axis=1).reshape(-1)
    zs = jnp.zeros((NNP, D), jnp.float32)

    p0, p1, u = _node_mm(h, wcat[0], bcat[0][None])
    for l in range(NL):
        ce = _edge_mm(e, wct[l], bC[l][None])
        ef, acc = _edge_pass(eix, zs, p0, p1, ce)
        e = ef
        if l + 1 < NL:
            p0, p1, u, h = _node_mm_fused(acc, u, h, wcat[l + 1],
                                          bcat[l + 1][None])
        else:
            h = _finish(acc, u, h)
    return (h, e)
